# baseline (speedup 1.0000x reference)
import functools

import jax
import jax.numpy as jnp
from jax import lax
from jax.experimental import pallas as pl
from jax.experimental.pallas import tpu as pltpu
from jax.experimental.pallas import tpu_sc as plsc

VOCAB = 1000000
D = 16
NI = 16384
NS = 50
B = NI * NS
NC = 2
NSUB = 16
NW = NC * NSUB
IPW = NI // NW
GPW = IPW * NS

BI = 16
NBLK = IPW // BI
G = BI * NS

_mesh = plsc.VectorSubcoreMesh(core_axis_name="c", subcore_axis_name="s")


@functools.partial(
    pl.kernel,
    mesh=_mesh,
    compiler_params=pltpu.CompilerParams(use_tc_tiling_on_sc=False),
    out_type=jax.ShapeDtypeStruct((NI, 56, 128), jnp.float32),
    scratch_types=[
        pltpu.VMEM((GPW,), jnp.int32),
        pltpu.VMEM((2, G, D), jnp.float32),
        pltpu.SemaphoreType.DMA((2,)),
        pltpu.SemaphoreType.DMA,
    ],
)
def _sc_lookup(idx_hbm, table_hbm, out_hbm, idx_v, rows_v, wsem, gsem):
    c = lax.axis_index("c")
    s = lax.axis_index("s")
    wid = s * NC + c
    i_base = wid * IPW

    pltpu.sync_copy(idx_hbm.at[pl.ds(wid * GPW, GPW)], idx_v)

    def block(b, _):
        buf = lax.rem(b, 2)
        i0 = i_base + b * BI

        @pl.when(b >= 2)
        def _():
            for j in range(BI):
                pltpu.make_async_copy(
                    rows_v.at[buf, pl.ds(j * NS, NS)],
                    out_hbm.at[i0 - 2 * BI + j, pl.ds(0, NS), pl.ds(0, D)],
                    wsem.at[buf]).wait()

        gather = pltpu.make_async_copy(
            table_hbm.at[idx_v.at[pl.ds(b * G, G)]], rows_v.at[buf], gsem)
        gather.start()
        gather.wait()
        for j in range(BI):
            pltpu.async_copy(
                rows_v.at[buf, pl.ds(j * NS, NS)],
                out_hbm.at[i0 + j, pl.ds(0, NS), pl.ds(0, D)], wsem.at[buf])

        return ()

    lax.fori_loop(0, NBLK, block, ())

    for bb in (NBLK - 2, NBLK - 1):
        for j in range(BI):
            pltpu.make_async_copy(
                rows_v.at[bb % 2, pl.ds(j * NS, NS)],
                out_hbm.at[i_base + bb * BI + j, pl.ds(0, NS), pl.ds(0, D)],
                wsem.at[bb % 2]).wait()


def kernel(indices, input_table):
    zero = lax.optimization_barrier(jnp.int32(0))
    flat_idx = (indices.astype(jnp.int32) + zero).reshape(B)
    eye = lax.optimization_barrier(jnp.eye(D, dtype=jnp.float32))
    table = lax.dot_general(
        input_table, eye, (((1,), (0,)), ((), ())),
        precision=lax.Precision.HIGHEST)
    out_p = _sc_lookup(flat_idx, table)
    return out_p[:, :NS, :D]


# speedup vs baseline: 1.9251x; 1.9251x over previous
import functools

import jax
import jax.numpy as jnp
from jax import lax
from jax.experimental import pallas as pl
from jax.experimental.pallas import tpu as pltpu
from jax.experimental.pallas import tpu_sc as plsc

VOCAB = 1000000
D = 16
NI = 16384
NS = 50
NC = 2
NSUB = 16
NW = NC * NSUB
IPW = NI // NW

BI = 16
NBLK = IPW // BI
G = BI * NS

_mesh = plsc.VectorSubcoreMesh(core_axis_name="c", subcore_axis_name="s")


@functools.partial(
    pl.kernel,
    mesh=_mesh,
    compiler_params=pltpu.CompilerParams(use_tc_tiling_on_sc=False),
    out_type=jax.ShapeDtypeStruct((NI, 56, 128), jnp.float32),
    scratch_types=[
        pltpu.VMEM((2, BI, 128), jnp.int32),
        pltpu.VMEM((2, BI, 56, D), jnp.float32),
        pltpu.SemaphoreType.DMA((2,)),
        pltpu.SemaphoreType.DMA((2,)),
        pltpu.SemaphoreType.DMA,
    ],
)
def _sc_lookup(idx_hbm, table_hbm, out_hbm, idx_v, rows_v, isem, wsem, gsem):
    c = lax.axis_index("c")
    s = lax.axis_index("s")
    wid = s * NC + c
    i_base = wid * IPW

    pltpu.async_copy(idx_hbm.at[pl.ds(i_base, BI)], idx_v.at[0], isem.at[0])

    def block(b, _):
        buf = lax.rem(b, 2)
        i0 = i_base + b * BI

        pltpu.make_async_copy(
            idx_hbm.at[pl.ds(i0, BI)], idx_v.at[buf], isem.at[buf]).wait()

        @pl.when(b >= 2)
        def _():
            for j in range(BI):
                pltpu.make_async_copy(
                    rows_v.at[buf, j, pl.ds(0, NS)],
                    out_hbm.at[i0 - 2 * BI + j, pl.ds(0, NS), pl.ds(0, D)],
                    wsem.at[buf]).wait()

        for j in range(BI):
            pltpu.async_copy(
                table_hbm.at[idx_v.at[buf, j, pl.ds(0, 56)]],
                rows_v.at[buf, j], gsem)

        @pl.when(b + 1 < NBLK)
        def _():
            pltpu.async_copy(
                idx_hbm.at[pl.ds(i0 + BI, BI)], idx_v.at[1 - buf],
                isem.at[1 - buf])

        for j in range(BI):
            pltpu.make_async_copy(
                table_hbm.at[idx_v.at[buf, j, pl.ds(0, 56)]],
                rows_v.at[buf, j], gsem).wait()
        for j in range(BI):
            pltpu.async_copy(
                rows_v.at[buf, j, pl.ds(0, NS)],
                out_hbm.at[i0 + j, pl.ds(0, NS), pl.ds(0, D)], wsem.at[buf])

        return ()

    lax.fori_loop(0, NBLK, block, ())

    for bb in (NBLK - 2, NBLK - 1):
        for j in range(BI):
            pltpu.make_async_copy(
                rows_v.at[bb % 2, j, pl.ds(0, NS)],
                out_hbm.at[i_base + bb * BI + j, pl.ds(0, NS), pl.ds(0, D)],
                wsem.at[bb % 2]).wait()


def kernel(indices, input_table):
    filler = (jnp.arange(NI, dtype=jnp.int32)[:, None] * 131
              + jnp.arange(128 - NS, dtype=jnp.int32)[None, :]) % VOCAB
    idx_p = jnp.concatenate([indices.astype(jnp.int32), filler], axis=1)
    out_p = _sc_lookup(idx_p, input_table)
    return out_p[:, :NS, :D]


# speedup vs baseline: 2.0025x; 1.0402x over previous
import functools

import jax
import jax.numpy as jnp
from jax import lax
from jax.experimental import pallas as pl
from jax.experimental.pallas import tpu as pltpu
from jax.experimental.pallas import tpu_sc as plsc

VOCAB = 1000000
D = 16
NI = 16384
NS = 50
B = NI * NS
NC = 2
NSUB = 16
NW = NC * NSUB
IPW = NI // NW
GPW = IPW * NS

BI = 32
NBLK = IPW // BI
G = BI * NS

_mesh = plsc.VectorSubcoreMesh(core_axis_name="c", subcore_axis_name="s")


@functools.partial(
    pl.kernel,
    mesh=_mesh,
    compiler_params=pltpu.CompilerParams(use_tc_tiling_on_sc=False),
    out_type=jax.ShapeDtypeStruct((NI, 56, 128), jnp.float32),
    scratch_types=[
        pltpu.VMEM((GPW,), jnp.int32),
        pltpu.VMEM((2, G, D), jnp.float32),
        pltpu.SemaphoreType.DMA((2,)),
        pltpu.SemaphoreType.DMA((4,)),
    ],
)
def _sc_lookup(idx_hbm, table_hbm, out_hbm, idx_v, rows_v, wsem, gsem):
    c = lax.axis_index("c")
    s = lax.axis_index("s")
    wid = s * NC + c
    i_base = wid * IPW

    pltpu.sync_copy(idx_hbm.at[pl.ds(wid * GPW, GPW)], idx_v)

    def block(b, _):
        buf = lax.rem(b, 2)
        i0 = i_base + b * BI

        @pl.when(b >= 2)
        def _():
            for j in range(BI):
                pltpu.make_async_copy(
                    rows_v.at[buf, pl.ds(j * NS, NS)],
                    out_hbm.at[i0 - 2 * BI + j, pl.ds(0, NS), pl.ds(0, D)],
                    wsem.at[buf]).wait()

        NQ = 4
        QI = BI // NQ
        QG = G // NQ
        for q in range(NQ):
            pltpu.async_copy(
                table_hbm.at[idx_v.at[pl.ds(b * G + q * QG, QG)]],
                rows_v.at[buf, pl.ds(q * QG, QG)], gsem.at[q])
        for q in range(NQ):
            pltpu.make_async_copy(
                table_hbm.at[idx_v.at[pl.ds(b * G + q * QG, QG)]],
                rows_v.at[buf, pl.ds(q * QG, QG)], gsem.at[q]).wait()
            for j in range(q * QI, (q + 1) * QI):
                pltpu.async_copy(
                    rows_v.at[buf, pl.ds(j * NS, NS)],
                    out_hbm.at[i0 + j, pl.ds(0, NS), pl.ds(0, D)],
                    wsem.at[buf])

        return ()

    lax.fori_loop(0, NBLK, block, ())

    for bb in (NBLK - 2, NBLK - 1):
        for j in range(BI):
            pltpu.make_async_copy(
                rows_v.at[bb % 2, pl.ds(j * NS, NS)],
                out_hbm.at[i_base + bb * BI + j, pl.ds(0, NS), pl.ds(0, D)],
                wsem.at[bb % 2]).wait()


def kernel(indices, input_table):
    zero = lax.optimization_barrier(jnp.int32(0))
    flat_idx = (indices.astype(jnp.int32) + zero).reshape(B)
    out_p = _sc_lookup(flat_idx, input_table)
    return out_p[:, :NS, :D]


# speedup vs baseline: 2.0033x; 1.0004x over previous
import functools

import jax
import jax.numpy as jnp
from jax import lax
from jax.experimental import pallas as pl
from jax.experimental.pallas import tpu as pltpu
from jax.experimental.pallas import tpu_sc as plsc

VOCAB = 1000000
D = 16
NI = 16384
NS = 50
B = NI * NS
NC = 2
NSUB = 16
NW = NC * NSUB
IPW = NI // NW
GPW = IPW * NS

BI = 32
NBLK = IPW // BI
G = BI * NS

_mesh = plsc.VectorSubcoreMesh(core_axis_name="c", subcore_axis_name="s")


@functools.partial(
    pl.kernel,
    mesh=_mesh,
    compiler_params=pltpu.CompilerParams(use_tc_tiling_on_sc=False),
    out_type=jax.ShapeDtypeStruct((NI, 56, 128), jnp.float32),
    scratch_types=[
        pltpu.VMEM((GPW,), jnp.int32),
        pltpu.VMEM((2, G, D), jnp.float32),
        pltpu.SemaphoreType.DMA((2,)),
        pltpu.SemaphoreType.DMA((8,)),
    ],
)
def _sc_lookup(idx_hbm, table_hbm, out_hbm, idx_v, rows_v, wsem, gsem):
    c = lax.axis_index("c")
    s = lax.axis_index("s")
    wid = s * NC + c
    i_base = wid * IPW

    pltpu.sync_copy(idx_hbm.at[pl.ds(wid * GPW, GPW)], idx_v)

    def block(b, _):
        buf = lax.rem(b, 2)
        i0 = i_base + b * BI

        @pl.when(b >= 2)
        def _():
            for j in range(BI):
                pltpu.make_async_copy(
                    rows_v.at[buf, pl.ds(j * NS, NS)],
                    out_hbm.at[i0 - 2 * BI + j, pl.ds(0, NS), pl.ds(0, D)],
                    wsem.at[buf]).wait()

        NQ = 8
        QI = BI // NQ
        QG = G // NQ
        for q in range(NQ):
            pltpu.async_copy(
                table_hbm.at[idx_v.at[pl.ds(b * G + q * QG, QG)]],
                rows_v.at[buf, pl.ds(q * QG, QG)], gsem.at[q])
        for q in range(NQ):
            pltpu.make_async_copy(
                table_hbm.at[idx_v.at[pl.ds(b * G + q * QG, QG)]],
                rows_v.at[buf, pl.ds(q * QG, QG)], gsem.at[q]).wait()
            for j in range(q * QI, (q + 1) * QI):
                pltpu.async_copy(
                    rows_v.at[buf, pl.ds(j * NS, NS)],
                    out_hbm.at[i0 + j, pl.ds(0, NS), pl.ds(0, D)],
                    wsem.at[buf])

        return ()

    lax.fori_loop(0, NBLK, block, ())

    for bb in (NBLK - 2, NBLK - 1):
        for j in range(BI):
            pltpu.make_async_copy(
                rows_v.at[bb % 2, pl.ds(j * NS, NS)],
                out_hbm.at[i_base + bb * BI + j, pl.ds(0, NS), pl.ds(0, D)],
                wsem.at[bb % 2]).wait()


def kernel(indices, input_table):
    zero = lax.optimization_barrier(jnp.int32(0))
    flat_idx = (indices.astype(jnp.int32) + zero).reshape(B)
    out_p = _sc_lookup(flat_idx, input_table)
    return out_p[:, :NS, :D]
